# baseline (speedup 1.0000x reference)
import functools
import math

import jax
import jax.numpy as jnp
from jax import lax
from jax.experimental import pallas as pl
from jax.experimental.pallas import tpu as pltpu
from jax.experimental.pallas import tpu_sc as plsc

NVERT = 100000
NFACE = 200000
STEPS = 10
DTC = 1e-08
GAM = 1.0
VTGT = 1.0
BULKC = 2500.0
PRES0 = 100.0
VTS = VTGT * math.exp(PRES0 / BULKC)

L = 16
NT = 16
NC = 2
NW = NC * NT
C = 128
D = 8
NCH = 52
FPW = NCH * C
NFP = NW * FPW
NVP = 100352
HALF = NVP // 2
TPS = HALF // NT
UCH = 224
UN = TPS // UCH
ICH = 28

_f32 = jnp.float32
_i32 = jnp.int32


def _lane_iota():
    return lax.broadcasted_iota(_i32, (L,), 0)


def _splat_i(val):
    return jnp.full((L,), val, dtype=_i32)


def _splat_f(val):
    return jnp.full((L,), val, dtype=_f32)


def _rsqrt16(x):
    i = plsc.bitcast(x, _i32)
    i = _splat_i(0x5F3759DF) - lax.shift_right_arithmetic(i, _splat_i(1))
    y = plsc.bitcast(i, _f32)
    half = _splat_f(0.5)
    three_half = _splat_f(1.5)
    hx = half * x
    for _ in range(3):
        y = y * (three_half - hx * y * y)
    return y


def _cross(ax, ay, az, bx, by, bz):
    return (ay * bz - az * by, az * bx - ax * bz, ax * by - ay * bx)


def _body(nsteps, vhbm, i0h, i1h, i2h,
          ohbm, vpart, xbuf, ybuf,
          vsh, fgsh,
          idxb, g0, g1, g2, r0, r1, r2,
          exb, vbu, fgmb, fgsb, zb, volstage, vpartb,
          gsem, ssem, isem, xsem):
    cid = lax.axis_index("c")
    sid = lax.axis_index("s")
    wid = sid * NC + cid
    peer = 1 - cid
    lane = _lane_iota()
    cols = [_splat_i(c) for c in range(6)]

    def xbarrier():
        plsc.subcore_barrier()
        pl.semaphore_signal(xsem, 1, core_index=peer)
        pl.semaphore_wait(xsem, 1)

    def idx_issue(j, s):
        pltpu.async_copy(i0h.at[wid, j], idxb.at[s, 0], isem)
        pltpu.async_copy(i1h.at[wid, j], idxb.at[s, 1], isem)
        pltpu.async_copy(i2h.at[wid, j], idxb.at[s, 2], isem)

    def idx_drain(j, s):
        pltpu.make_async_copy(i0h.at[wid, j], idxb.at[s, 0], isem).wait()
        pltpu.make_async_copy(i1h.at[wid, j], idxb.at[s, 1], isem).wait()
        pltpu.make_async_copy(i2h.at[wid, j], idxb.at[s, 2], isem).wait()

    def gather_issue(s2, s4):
        pltpu.async_copy(vsh.at[idxb.at[s4, 0]], g0.at[s2], gsem)
        pltpu.async_copy(vsh.at[idxb.at[s4, 1]], g1.at[s2], gsem)
        pltpu.async_copy(vsh.at[idxb.at[s4, 2]], g2.at[s2], gsem)

    def gather_drain(s2, s4):
        pltpu.make_async_copy(vsh.at[idxb.at[s4, 0]], g0.at[s2], gsem).wait()
        pltpu.make_async_copy(vsh.at[idxb.at[s4, 1]], g1.at[s2], gsem).wait()
        pltpu.make_async_copy(vsh.at[idxb.at[s4, 2]], g2.at[s2], gsem).wait()

    def scat_issue(s2, s4):
        pltpu.async_copy(r0.at[s2], fgsh.at[idxb.at[s4, 0]], ssem, add=True)
        pltpu.async_copy(r1.at[s2], fgsh.at[idxb.at[s4, 1]], ssem, add=True)
        pltpu.async_copy(r2.at[s2], fgsh.at[idxb.at[s4, 2]], ssem, add=True)

    def scat_drain(s2, s4):
        pltpu.make_async_copy(r0.at[s2], fgsh.at[idxb.at[s4, 0]], ssem).wait()
        pltpu.make_async_copy(r1.at[s2], fgsh.at[idxb.at[s4, 1]], ssem).wait()
        pltpu.make_async_copy(r2.at[s2], fgsh.at[idxb.at[s4, 2]], ssem).wait()

    zero = _splat_f(0.0)

    def zb_zero(g, _):
        tt = lane + g * _splat_i(L)
        row = lax.div(tt, _splat_i(D))
        colv = tt - row * _splat_i(D)
        plsc.store_scatter(zb, [row, colv], zero)
        return _

    lax.fori_loop(0, UCH * D // L, zb_zero, None)
    for s2 in range(2):
        for g in range(C * 2 // L):
            tt = lane + _splat_i(g * L)
            row = lax.shift_right_arithmetic(tt, _splat_i(1))
            colv = (tt & _splat_i(1)) + _splat_i(6)
            plsc.store_scatter(r0.at[s2], [row, colv], zero)
            plsc.store_scatter(r1.at[s2], [row, colv], zero)
            plsc.store_scatter(r2.at[s2], [row, colv], zero)

    def init_slices(r, _):
        rows = pl.ds(sid * (ICH * UCH) + r * UCH, UCH)
        pltpu.sync_copy(vhbm.at[rows], vbu)
        pltpu.sync_copy(vbu, vsh.at[rows])
        pltpu.sync_copy(zb, fgsh.at[rows])
        return _

    lax.fori_loop(0, ICH, init_slices, None)
    idx_issue(0, 0)
    idx_issue(1, 1)
    plsc.subcore_barrier()

    neg_half_gam_dt = _splat_f(-0.5 * GAM * DTC)

    def step_body(_, carry):
        idx_drain(0, 0)
        gather_issue(0, 0)

        def chunk_body(j, vol_acc):
            s2 = lax.rem(j, 2)
            s4 = lax.rem(j, 4)

            @pl.when(j < NCH - 2)
            def _():
                idx_issue(j + 2, lax.rem(j + 2, 4))

            @pl.when(j < NCH - 1)
            def _():
                idx_drain(j + 1, lax.rem(j + 1, 4))

            gather_drain(s2, s4)

            @pl.when(j < NCH - 1)
            def _():
                gather_issue(lax.rem(j + 1, 2), lax.rem(j + 1, 4))

            for g in range(C // L):
                rowi = lane + _splat_i(g * L)
                v0x = plsc.load_gather(g0.at[s2], [rowi, cols[0]])
                v0y = plsc.load_gather(g0.at[s2], [rowi, cols[1]])
                v0z = plsc.load_gather(g0.at[s2], [rowi, cols[2]])
                v1x = plsc.load_gather(g1.at[s2], [rowi, cols[0]])
                v1y = plsc.load_gather(g1.at[s2], [rowi, cols[1]])
                v1z = plsc.load_gather(g1.at[s2], [rowi, cols[2]])
                v2x = plsc.load_gather(g2.at[s2], [rowi, cols[0]])
                v2y = plsc.load_gather(g2.at[s2], [rowi, cols[1]])
                v2z = plsc.load_gather(g2.at[s2], [rowi, cols[2]])
                c0x, c0y, c0z = _cross(v1x, v1y, v1z, v2x, v2y, v2z)
                c1x, c1y, c1z = _cross(v2x, v2y, v2z, v0x, v0y, v0z)
                c2x, c2y, c2z = _cross(v0x, v0y, v0z, v1x, v1y, v1z)
                nx = c0x + c1x + c2x
                ny = c0y + c1y + c2y
                nz = c0z + c1z + c2z
                nn = nx * nx + ny * ny + nz * nz + _splat_f(1e-20)
                rinv = _rsqrt16(nn)
                nhx, nhy, nhz = nx * rinv, ny * rinv, nz * rinv
                e0x, e0y, e0z = v2x - v1x, v2y - v1y, v2z - v1z
                e1x, e1y, e1z = v0x - v2x, v0y - v2y, v0z - v2z
                e2x, e2y, e2z = v1x - v0x, v1y - v0y, v1z - v0z
                t0x, t0y, t0z = _cross(nhx, nhy, nhz, e0x, e0y, e0z)
                t1x, t1y, t1z = _cross(nhx, nhy, nhz, e1x, e1y, e1z)
                t2x, t2y, t2z = _cross(nhx, nhy, nhz, e2x, e2y, e2z)
                plsc.store_scatter(r0.at[s2], [rowi, cols[0]], t0x * neg_half_gam_dt)
                plsc.store_scatter(r0.at[s2], [rowi, cols[1]], t0y * neg_half_gam_dt)
                plsc.store_scatter(r0.at[s2], [rowi, cols[2]], t0z * neg_half_gam_dt)
                plsc.store_scatter(r0.at[s2], [rowi, cols[3]], c0x)
                plsc.store_scatter(r0.at[s2], [rowi, cols[4]], c0y)
                plsc.store_scatter(r0.at[s2], [rowi, cols[5]], c0z)
                plsc.store_scatter(r1.at[s2], [rowi, cols[0]], t1x * neg_half_gam_dt)
                plsc.store_scatter(r1.at[s2], [rowi, cols[1]], t1y * neg_half_gam_dt)
                plsc.store_scatter(r1.at[s2], [rowi, cols[2]], t1z * neg_half_gam_dt)
                plsc.store_scatter(r1.at[s2], [rowi, cols[3]], c1x)
                plsc.store_scatter(r1.at[s2], [rowi, cols[4]], c1y)
                plsc.store_scatter(r1.at[s2], [rowi, cols[5]], c1z)
                plsc.store_scatter(r2.at[s2], [rowi, cols[0]], t2x * neg_half_gam_dt)
                plsc.store_scatter(r2.at[s2], [rowi, cols[1]], t2y * neg_half_gam_dt)
                plsc.store_scatter(r2.at[s2], [rowi, cols[2]], t2z * neg_half_gam_dt)
                plsc.store_scatter(r2.at[s2], [rowi, cols[3]], c2x)
                plsc.store_scatter(r2.at[s2], [rowi, cols[4]], c2y)
                plsc.store_scatter(r2.at[s2], [rowi, cols[5]], c2z)
                vol_acc = vol_acc + v0x * c0x + v0y * c0y + v0z * c0z

            @pl.when(j >= 1)
            def _():
                scat_drain(lax.rem(j + 1, 2), lax.rem(j + 3, 4))

            scat_issue(s2, s4)
            return vol_acc

        vol_acc = lax.fori_loop(0, NCH, chunk_body, _splat_f(0.0))
        scat_drain((NCH - 1) % 2, (NCH - 1) % 4)
        plsc.subcore_barrier()

        volstage[...] = vol_acc
        pltpu.sync_copy(volstage, vpart.at[cid, sid])
        ot_base = peer * HALF + sid * TPS

        pltpu.sync_copy(fgsh.at[pl.ds(ot_base, UCH)], exb.at[0])

        def export_fg(u, _):
            s2 = lax.rem(u, 2)
            s1 = lax.rem(u + 1, 2)
            rows = pl.ds(ot_base + u * UCH, UCH)
            hrows = pl.ds(sid * TPS + u * UCH, UCH)

            @pl.when(u >= 1)
            def _():
                hprev = pl.ds(sid * TPS + (u - 1) * UCH, UCH)
                pltpu.make_async_copy(
                    exb.at[s1], xbuf.at[cid, hprev], isem).wait()

            pltpu.async_copy(exb.at[s2], xbuf.at[cid, hrows], isem)

            @pl.when(u < UN - 1)
            def _():
                nrows = pl.ds(ot_base + (u + 1) * UCH, UCH)
                pltpu.sync_copy(fgsh.at[nrows], exb.at[s1])

            pltpu.sync_copy(zb, fgsh.at[rows])
            return _

        lax.fori_loop(0, UN, export_fg, None)
        pltpu.make_async_copy(
            exb.at[(UN - 1) % 2],
            xbuf.at[cid, pl.ds(sid * TPS + (UN - 1) * UCH, UCH)],
            isem).wait()
        xbarrier()

        pltpu.sync_copy(vpart, vpartb)
        acc = vpartb[0, 0, :]
        for ci in range(NC):
            for i in range(NT):
                if ci == 0 and i == 0:
                    continue
                acc = acc + vpartb[ci, i, :]
        tot16 = jnp.full((L,), jnp.sum(acc), dtype=_f32)
        kfac = BULKC * DTC / (6.0 * VTS)
        pfac = (_splat_f(VTS) - tot16 * _splat_f(1.0 / 6.0)) * _splat_f(kfac)

        my_base = cid * HALF + sid * TPS

        def upd_body(u, _):
            rows = pl.ds(my_base + u * UCH, UCH)
            hrows = pl.ds(sid * TPS + u * UCH, UCH)
            pltpu.async_copy(vsh.at[rows], vbu, gsem)
            pltpu.async_copy(fgsh.at[rows], fgmb, ssem)
            pltpu.async_copy(xbuf.at[peer, hrows], fgsb, isem)
            pltpu.make_async_copy(vsh.at[rows], vbu, gsem).wait()
            pltpu.make_async_copy(fgsh.at[rows], fgmb, ssem).wait()
            pltpu.make_async_copy(xbuf.at[peer, hrows], fgsb, isem).wait()

            def upd_group(gq, _2):
                for q in range(4):
                    tt = lane + (gq * _splat_i(4 * L) + _splat_i(q * L))
                    row = lax.div(tt, _splat_i(3))
                    colv = tt - row * _splat_i(3)
                    colg = colv + _splat_i(3)
                    v = plsc.load_gather(vbu, [row, colv])
                    fm = plsc.load_gather(fgmb, [row, colv])
                    fs = plsc.load_gather(fgsb, [row, colv])
                    gm = plsc.load_gather(fgmb, [row, colg])
                    gs = plsc.load_gather(fgsb, [row, colg])
                    out = v + fm + fs + pfac * (gm + gs)
                    plsc.store_scatter(vbu, [row, colv], out)
                return _2

            lax.fori_loop(0, UCH * 3 // L // 4, upd_group, None)
            pltpu.sync_copy(vbu, vsh.at[rows])
            pltpu.sync_copy(vbu, ybuf.at[cid, hrows])
            pltpu.sync_copy(zb, fgsh.at[rows])
            return _

        lax.fori_loop(0, UN, upd_body, None)
        xbarrier()

        pltpu.async_copy(
            ybuf.at[peer, pl.ds(sid * TPS, UCH)], exb.at[0], isem)

        def import_v(u, _):
            s2 = lax.rem(u, 2)
            s1 = lax.rem(u + 1, 2)
            rows = pl.ds(ot_base + u * UCH, UCH)
            hrows = pl.ds(sid * TPS + u * UCH, UCH)
            pltpu.make_async_copy(
                ybuf.at[peer, hrows], exb.at[s2], isem).wait()

            @pl.when(u < UN - 1)
            def _():
                hnext = pl.ds(sid * TPS + (u + 1) * UCH, UCH)
                pltpu.async_copy(ybuf.at[peer, hnext], exb.at[s1], isem)

            pltpu.sync_copy(exb.at[s2], vsh.at[rows])
            return _

        lax.fori_loop(0, UN, import_v, None)
        idx_issue(0, 0)
        idx_issue(1, 1)
        plsc.subcore_barrier()
        return carry

    lax.fori_loop(0, nsteps, step_body, 0)
    idx_drain(0, 0)
    idx_drain(1, 1)

    @pl.when(cid == 0)
    def _():
        def write_out(r, _):
            rows = pl.ds(sid * (ICH * UCH) + r * UCH, UCH)
            pltpu.sync_copy(vsh.at[rows], vbu)
            pltpu.sync_copy(vbu, ohbm.at[rows])
            return _

        lax.fori_loop(0, ICH, write_out, None)


def _make_kernel(nsteps=STEPS):
    mesh = plsc.VectorSubcoreMesh(
        core_axis_name="c", subcore_axis_name="s", num_cores=NC,
        num_subcores=NT,
    )
    return pl.kernel(
        functools.partial(_body, nsteps),
        out_type=(
            jax.ShapeDtypeStruct((NVP, D), _f32),
            jax.ShapeDtypeStruct((NC, NT, L), _f32),
            jax.ShapeDtypeStruct((NC, HALF, D), _f32),
            jax.ShapeDtypeStruct((NC, HALF, D), _f32),
        ),
        mesh=mesh,
        compiler_params=pltpu.CompilerParams(
            needs_layout_passes=False, use_tc_tiling_on_sc=False
        ),
        scratch_types=[
            pltpu.VMEM_SHARED((NVP, D), _f32),
            pltpu.VMEM_SHARED((NVP, D), _f32),
            pltpu.VMEM((4, 3, C), _i32),
            pltpu.VMEM((2, C, D), _f32),
            pltpu.VMEM((2, C, D), _f32),
            pltpu.VMEM((2, C, D), _f32),
            pltpu.VMEM((2, C, D), _f32),
            pltpu.VMEM((2, C, D), _f32),
            pltpu.VMEM((2, C, D), _f32),
            pltpu.VMEM((2, UCH, D), _f32),
            pltpu.VMEM((UCH, D), _f32),
            pltpu.VMEM((UCH, D), _f32),
            pltpu.VMEM((UCH, D), _f32),
            pltpu.VMEM((UCH, D), _f32),
            pltpu.VMEM((L,), _f32),
            pltpu.VMEM((NC, NT, L), _f32),
            pltpu.SemaphoreType.DMA,
            pltpu.SemaphoreType.DMA,
            pltpu.SemaphoreType.DMA,
            pltpu.SemaphoreType.REGULAR,
        ],
    )


def kernel(vertices, faces):
    fc = faces.astype(_i32)
    pad_f = NFP - NFACE
    i0 = jnp.pad(fc[:, 0], (0, pad_f)).reshape(NW, NCH, C)
    i1 = jnp.pad(fc[:, 1], (0, pad_f)).reshape(NW, NCH, C)
    i2 = jnp.pad(fc[:, 2], (0, pad_f)).reshape(NW, NCH, C)
    vp = jnp.pad(vertices.astype(_f32), ((0, NVP - NVERT), (0, D - 3)))
    out, _, _, _ = _make_kernel()(vp, i0, i1, i2)
    return (out[:NVERT, :3], faces)


# speedup vs baseline: 1.0496x; 1.0496x over previous
import functools
import math

import jax
import jax.numpy as jnp
from jax import lax
from jax.experimental import pallas as pl
from jax.experimental.pallas import tpu as pltpu
from jax.experimental.pallas import tpu_sc as plsc

NVERT = 100000
NFACE = 200000
STEPS = 10
DTC = 1e-08
GAM = 1.0
VTGT = 1.0
BULKC = 2500.0
PRES0 = 100.0
VTS = VTGT * math.exp(PRES0 / BULKC)

L = 16
NT = 16
NC = 2
NW = NC * NT
C = 128
D = 8
NCH = 52
FPW = NCH * C
NFP = NW * FPW
NVP = 100352
HALF = NVP // 2
TPS = HALF // NT
UCH = 448
UN = TPS // UCH
ICH = 14

_f32 = jnp.float32
_i32 = jnp.int32


def _lane_iota():
    return lax.broadcasted_iota(_i32, (L,), 0)


def _splat_i(val):
    return jnp.full((L,), val, dtype=_i32)


def _splat_f(val):
    return jnp.full((L,), val, dtype=_f32)


def _rsqrt16(x):
    i = plsc.bitcast(x, _i32)
    i = _splat_i(0x5F3759DF) - lax.shift_right_arithmetic(i, _splat_i(1))
    y = plsc.bitcast(i, _f32)
    half = _splat_f(0.5)
    three_half = _splat_f(1.5)
    hx = half * x
    for _ in range(3):
        y = y * (three_half - hx * y * y)
    return y


def _cross(ax, ay, az, bx, by, bz):
    return (ay * bz - az * by, az * bx - ax * bz, ax * by - ay * bx)


def _body(nsteps, vhbm, i0h, i1h, i2h,
          ohbm, vpart, xbuf, ybuf,
          vsh, fgsh,
          idxb, g0, g1, g2, r0, r1, r2,
          vbu, fgmb, fgsb, zb, volstage, vpartb,
          gsem, ssem, isem, xsem):
    cid = lax.axis_index("c")
    sid = lax.axis_index("s")
    wid = sid * NC + cid
    peer = 1 - cid
    lane = _lane_iota()
    cols = [_splat_i(c) for c in range(6)]

    def xbarrier():
        plsc.subcore_barrier()
        pl.semaphore_signal(xsem, 1, core_index=peer)
        pl.semaphore_wait(xsem, 1)

    def idx_issue(j, s):
        pltpu.async_copy(i0h.at[wid, j], idxb.at[s, 0], isem)
        pltpu.async_copy(i1h.at[wid, j], idxb.at[s, 1], isem)
        pltpu.async_copy(i2h.at[wid, j], idxb.at[s, 2], isem)

    def idx_drain(j, s):
        pltpu.make_async_copy(i0h.at[wid, j], idxb.at[s, 0], isem).wait()
        pltpu.make_async_copy(i1h.at[wid, j], idxb.at[s, 1], isem).wait()
        pltpu.make_async_copy(i2h.at[wid, j], idxb.at[s, 2], isem).wait()

    def gather_issue(s2, s4):
        pltpu.async_copy(vsh.at[idxb.at[s4, 0]], g0.at[s2], gsem)
        pltpu.async_copy(vsh.at[idxb.at[s4, 1]], g1.at[s2], gsem)
        pltpu.async_copy(vsh.at[idxb.at[s4, 2]], g2.at[s2], gsem)

    def gather_drain(s2, s4):
        pltpu.make_async_copy(vsh.at[idxb.at[s4, 0]], g0.at[s2], gsem).wait()
        pltpu.make_async_copy(vsh.at[idxb.at[s4, 1]], g1.at[s2], gsem).wait()
        pltpu.make_async_copy(vsh.at[idxb.at[s4, 2]], g2.at[s2], gsem).wait()

    def scat_issue(s2, s4):
        pltpu.async_copy(r0.at[s2], fgsh.at[idxb.at[s4, 0]], ssem, add=True)
        pltpu.async_copy(r1.at[s2], fgsh.at[idxb.at[s4, 1]], ssem, add=True)
        pltpu.async_copy(r2.at[s2], fgsh.at[idxb.at[s4, 2]], ssem, add=True)

    def scat_drain(s2, s4):
        pltpu.make_async_copy(r0.at[s2], fgsh.at[idxb.at[s4, 0]], ssem).wait()
        pltpu.make_async_copy(r1.at[s2], fgsh.at[idxb.at[s4, 1]], ssem).wait()
        pltpu.make_async_copy(r2.at[s2], fgsh.at[idxb.at[s4, 2]], ssem).wait()

    zero = _splat_f(0.0)

    def zb_zero(g, _):
        tt = lane + g * _splat_i(L)
        row = lax.div(tt, _splat_i(D))
        colv = tt - row * _splat_i(D)
        plsc.store_scatter(zb, [row, colv], zero)
        return _

    lax.fori_loop(0, UCH * D // L, zb_zero, None)
    for s2 in range(2):
        for g in range(C * 2 // L):
            tt = lane + _splat_i(g * L)
            row = lax.shift_right_arithmetic(tt, _splat_i(1))
            colv = (tt & _splat_i(1)) + _splat_i(6)
            plsc.store_scatter(r0.at[s2], [row, colv], zero)
            plsc.store_scatter(r1.at[s2], [row, colv], zero)
            plsc.store_scatter(r2.at[s2], [row, colv], zero)

    def init_slices(r, _):
        rows = pl.ds(sid * (ICH * UCH) + r * UCH, UCH)
        pltpu.sync_copy(vhbm.at[rows], vbu)
        pltpu.sync_copy(vbu, vsh.at[rows])
        pltpu.sync_copy(zb, fgsh.at[rows])
        return _

    lax.fori_loop(0, ICH, init_slices, None)
    idx_issue(0, 0)
    idx_issue(1, 1)
    plsc.subcore_barrier()

    neg_half_gam_dt = _splat_f(-0.5 * GAM * DTC)

    def step_body(_, carry):
        idx_drain(0, 0)
        gather_issue(0, 0)

        def chunk_body(j, vol_acc):
            s2 = lax.rem(j, 2)
            s4 = lax.rem(j, 4)

            @pl.when(j < NCH - 1)
            def _():
                idx_drain(j + 1, lax.rem(j + 1, 4))

            jn = lax.rem(j + 2, NCH)
            idx_issue(jn, lax.rem(jn, 4))

            gather_drain(s2, s4)

            @pl.when(j < NCH - 1)
            def _():
                gather_issue(lax.rem(j + 1, 2), lax.rem(j + 1, 4))

            for g in range(C // L):
                rowi = lane + _splat_i(g * L)
                v0x = plsc.load_gather(g0.at[s2], [rowi, cols[0]])
                v0y = plsc.load_gather(g0.at[s2], [rowi, cols[1]])
                v0z = plsc.load_gather(g0.at[s2], [rowi, cols[2]])
                v1x = plsc.load_gather(g1.at[s2], [rowi, cols[0]])
                v1y = plsc.load_gather(g1.at[s2], [rowi, cols[1]])
                v1z = plsc.load_gather(g1.at[s2], [rowi, cols[2]])
                v2x = plsc.load_gather(g2.at[s2], [rowi, cols[0]])
                v2y = plsc.load_gather(g2.at[s2], [rowi, cols[1]])
                v2z = plsc.load_gather(g2.at[s2], [rowi, cols[2]])
                c0x, c0y, c0z = _cross(v1x, v1y, v1z, v2x, v2y, v2z)
                c1x, c1y, c1z = _cross(v2x, v2y, v2z, v0x, v0y, v0z)
                c2x, c2y, c2z = _cross(v0x, v0y, v0z, v1x, v1y, v1z)
                nx = c0x + c1x + c2x
                ny = c0y + c1y + c2y
                nz = c0z + c1z + c2z
                nn = nx * nx + ny * ny + nz * nz + _splat_f(1e-20)
                rinv = _rsqrt16(nn)
                nhx, nhy, nhz = nx * rinv, ny * rinv, nz * rinv
                e0x, e0y, e0z = v2x - v1x, v2y - v1y, v2z - v1z
                e1x, e1y, e1z = v0x - v2x, v0y - v2y, v0z - v2z
                e2x, e2y, e2z = v1x - v0x, v1y - v0y, v1z - v0z
                t0x, t0y, t0z = _cross(nhx, nhy, nhz, e0x, e0y, e0z)
                t1x, t1y, t1z = _cross(nhx, nhy, nhz, e1x, e1y, e1z)
                t2x, t2y, t2z = _cross(nhx, nhy, nhz, e2x, e2y, e2z)
                plsc.store_scatter(r0.at[s2], [rowi, cols[0]], t0x * neg_half_gam_dt)
                plsc.store_scatter(r0.at[s2], [rowi, cols[1]], t0y * neg_half_gam_dt)
                plsc.store_scatter(r0.at[s2], [rowi, cols[2]], t0z * neg_half_gam_dt)
                plsc.store_scatter(r0.at[s2], [rowi, cols[3]], c0x)
                plsc.store_scatter(r0.at[s2], [rowi, cols[4]], c0y)
                plsc.store_scatter(r0.at[s2], [rowi, cols[5]], c0z)
                plsc.store_scatter(r1.at[s2], [rowi, cols[0]], t1x * neg_half_gam_dt)
                plsc.store_scatter(r1.at[s2], [rowi, cols[1]], t1y * neg_half_gam_dt)
                plsc.store_scatter(r1.at[s2], [rowi, cols[2]], t1z * neg_half_gam_dt)
                plsc.store_scatter(r1.at[s2], [rowi, cols[3]], c1x)
                plsc.store_scatter(r1.at[s2], [rowi, cols[4]], c1y)
                plsc.store_scatter(r1.at[s2], [rowi, cols[5]], c1z)
                plsc.store_scatter(r2.at[s2], [rowi, cols[0]], t2x * neg_half_gam_dt)
                plsc.store_scatter(r2.at[s2], [rowi, cols[1]], t2y * neg_half_gam_dt)
                plsc.store_scatter(r2.at[s2], [rowi, cols[2]], t2z * neg_half_gam_dt)
                plsc.store_scatter(r2.at[s2], [rowi, cols[3]], c2x)
                plsc.store_scatter(r2.at[s2], [rowi, cols[4]], c2y)
                plsc.store_scatter(r2.at[s2], [rowi, cols[5]], c2z)
                vol_acc = vol_acc + v0x * c0x + v0y * c0y + v0z * c0z

            @pl.when(j >= 1)
            def _():
                scat_drain(lax.rem(j + 1, 2), lax.rem(j + 3, 4))

            scat_issue(s2, s4)
            return vol_acc

        vol_acc = lax.fori_loop(0, NCH, chunk_body, _splat_f(0.0))
        scat_drain((NCH - 1) % 2, (NCH - 1) % 4)
        plsc.subcore_barrier()

        volstage[...] = vol_acc
        pltpu.sync_copy(volstage, vpart.at[cid, sid])
        ot_base = peer * HALF + sid * TPS

        def export_fg(u, _):
            rows = pl.ds(ot_base + u * UCH, UCH)
            hrows = pl.ds(sid * TPS + u * UCH, UCH)
            pltpu.sync_copy(fgsh.at[rows], fgmb)
            pltpu.sync_copy(fgmb, xbuf.at[cid, hrows])
            pltpu.sync_copy(zb, fgsh.at[rows])
            return _

        lax.fori_loop(0, UN, export_fg, None)
        xbarrier()

        pltpu.sync_copy(vpart, vpartb)
        acc = vpartb[0, 0, :]
        for ci in range(NC):
            for i in range(NT):
                if ci == 0 and i == 0:
                    continue
                acc = acc + vpartb[ci, i, :]
        tot16 = jnp.full((L,), jnp.sum(acc), dtype=_f32)
        kfac = BULKC * DTC / (6.0 * VTS)
        pfac = (_splat_f(VTS) - tot16 * _splat_f(1.0 / 6.0)) * _splat_f(kfac)

        my_base = cid * HALF + sid * TPS

        def upd_body(u, _):
            rows = pl.ds(my_base + u * UCH, UCH)
            hrows = pl.ds(sid * TPS + u * UCH, UCH)
            pltpu.sync_copy(vsh.at[rows], vbu)
            pltpu.sync_copy(fgsh.at[rows], fgmb)
            pltpu.sync_copy(xbuf.at[peer, hrows], fgsb)

            def upd_group(gq, _2):
                for q in range(4):
                    tt = lane + (gq * _splat_i(4 * L) + _splat_i(q * L))
                    row = lax.div(tt, _splat_i(3))
                    colv = tt - row * _splat_i(3)
                    colg = colv + _splat_i(3)
                    v = plsc.load_gather(vbu, [row, colv])
                    fm = plsc.load_gather(fgmb, [row, colv])
                    fs = plsc.load_gather(fgsb, [row, colv])
                    gm = plsc.load_gather(fgmb, [row, colg])
                    gs = plsc.load_gather(fgsb, [row, colg])
                    out = v + fm + fs + pfac * (gm + gs)
                    plsc.store_scatter(vbu, [row, colv], out)
                return _2

            lax.fori_loop(0, UCH * 3 // L // 4, upd_group, None)
            pltpu.sync_copy(vbu, vsh.at[rows])
            pltpu.sync_copy(vbu, ybuf.at[cid, hrows])
            pltpu.sync_copy(zb, fgsh.at[rows])
            return _

        lax.fori_loop(0, UN, upd_body, None)
        xbarrier()

        def import_v(u, _):
            rows = pl.ds(ot_base + u * UCH, UCH)
            hrows = pl.ds(sid * TPS + u * UCH, UCH)
            pltpu.sync_copy(ybuf.at[peer, hrows], vbu)
            pltpu.sync_copy(vbu, vsh.at[rows])
            return _

        lax.fori_loop(0, UN, import_v, None)
        plsc.subcore_barrier()
        return carry

    lax.fori_loop(0, nsteps, step_body, 0)
    idx_drain(0, 0)
    idx_drain(1, 1)

    @pl.when(cid == 0)
    def _():
        def write_out(r, _):
            rows = pl.ds(sid * (ICH * UCH) + r * UCH, UCH)
            pltpu.sync_copy(vsh.at[rows], vbu)
            pltpu.sync_copy(vbu, ohbm.at[rows])
            return _

        lax.fori_loop(0, ICH, write_out, None)


def _make_kernel(nsteps=STEPS):
    mesh = plsc.VectorSubcoreMesh(
        core_axis_name="c", subcore_axis_name="s", num_cores=NC,
        num_subcores=NT,
    )
    return pl.kernel(
        functools.partial(_body, nsteps),
        out_type=(
            jax.ShapeDtypeStruct((NVP, D), _f32),
            jax.ShapeDtypeStruct((NC, NT, L), _f32),
            jax.ShapeDtypeStruct((NC, HALF, D), _f32),
            jax.ShapeDtypeStruct((NC, HALF, D), _f32),
        ),
        mesh=mesh,
        compiler_params=pltpu.CompilerParams(
            needs_layout_passes=False, use_tc_tiling_on_sc=False
        ),
        scratch_types=[
            pltpu.VMEM_SHARED((NVP, D), _f32),
            pltpu.VMEM_SHARED((NVP, D), _f32),
            pltpu.VMEM((4, 3, C), _i32),
            pltpu.VMEM((2, C, D), _f32),
            pltpu.VMEM((2, C, D), _f32),
            pltpu.VMEM((2, C, D), _f32),
            pltpu.VMEM((2, C, D), _f32),
            pltpu.VMEM((2, C, D), _f32),
            pltpu.VMEM((2, C, D), _f32),
            pltpu.VMEM((UCH, D), _f32),
            pltpu.VMEM((UCH, D), _f32),
            pltpu.VMEM((UCH, D), _f32),
            pltpu.VMEM((UCH, D), _f32),
            pltpu.VMEM((L,), _f32),
            pltpu.VMEM((NC, NT, L), _f32),
            pltpu.SemaphoreType.DMA,
            pltpu.SemaphoreType.DMA,
            pltpu.SemaphoreType.DMA,
            pltpu.SemaphoreType.REGULAR,
        ],
    )


def kernel(vertices, faces):
    fc = faces.astype(_i32)
    pad_f = NFP - NFACE
    i0 = jnp.pad(fc[:, 0], (0, pad_f)).reshape(NW, NCH, C)
    i1 = jnp.pad(fc[:, 1], (0, pad_f)).reshape(NW, NCH, C)
    i2 = jnp.pad(fc[:, 2], (0, pad_f)).reshape(NW, NCH, C)
    vp = jnp.pad(vertices.astype(_f32), ((0, NVP - NVERT), (0, D - 3)))
    out, _, _, _ = _make_kernel()(vp, i0, i1, i2)
    return (out[:NVERT, :3], faces)
